# baseline (speedup 1.0000x reference)
import functools

import jax
import jax.numpy as jnp
from jax import lax
from jax.experimental import pallas as pl
from jax.experimental.pallas import tpu as pltpu

B, H, S, D = 8, 32, 2048, 128
SBLK = 512


def _tc_body(pos_ref, cur_ref, out_ref):
    j = pl.program_id(2)
    pos = pos_ref[0]
    row = lax.broadcasted_iota(jnp.int32, (SBLK, D), 0) + j * SBLK
    out_ref[0, 0] = jnp.where(row == pos, cur_ref[0, 0], jnp.zeros((SBLK, D), jnp.float32))


@jax.jit
def kernel(cache, cur, dim, idx):
    del cache
    pos = (idx.astype(jnp.int32) - 1) + (jnp.asarray(dim, jnp.int32) - 2)

    grid_spec = pltpu.PrefetchScalarGridSpec(
        num_scalar_prefetch=1,
        grid=(B, H, S // SBLK),
        in_specs=[
            pl.BlockSpec((1, 1, 1, D), lambda b, h, j, pos_ref: (b, h, 0, 0)),
        ],
        out_specs=pl.BlockSpec((1, 1, SBLK, D), lambda b, h, j, pos_ref: (b, h, j, 0)),
    )
    out = pl.pallas_call(
        _tc_body,
        grid_spec=grid_spec,
        out_shape=jax.ShapeDtypeStruct((B, H, S, D), jnp.float32),
    )(pos, cur)
    return out


# speedup vs baseline: 4.6904x; 4.6904x over previous
import jax
import jax.numpy as jnp
from jax import lax
from jax.experimental import pallas as pl
from jax.experimental.pallas import tpu as pltpu

B, H, S, D = 8, 32, 2048, 128
BH = B * H
NBUF = 4
BPD = 2
PER_STEP = NBUF * BPD
NSTEP = BH // PER_STEP


def _tc_body(pos_ref, cur_ref, out_ref, z0, z1, z2, z3, s0, s1, s2, s3):
    i = pl.program_id(0)
    pos = pos_ref[0]
    zbufs = (z0, z1, z2, z3)
    sems = (s0, s1, s2, s3)

    @pl.when(i == 0)
    def _init():
        for q in range(NBUF):
            zbufs[q][...] = jnp.zeros((BPD, S, D), jnp.float32)

    for q in range(NBUF):
        base = i * PER_STEP + q * BPD

        @pl.when(i > 0)
        def _wait(q=q, base=base):
            pltpu.make_async_copy(
                zbufs[q], out_ref.at[pl.ds(base, BPD)], sems[q]).wait()

        for b in range(BPD):
            zbufs[q][b, pl.ds(pos, 1), :] = cur_ref[pl.ds(base + b, 1), 0, :]
        pltpu.make_async_copy(
            zbufs[q], out_ref.at[pl.ds(base, BPD)], sems[q]).start()

    @pl.when(i == NSTEP - 1)
    def _drain():
        for q in range(NBUF):
            base = i * PER_STEP + q * BPD
            pltpu.make_async_copy(
                zbufs[q], out_ref.at[pl.ds(base, BPD)], sems[q]).wait()


@jax.jit
def kernel(cache, cur, dim, idx):
    del cache
    pos = (idx.astype(jnp.int32) - 1) + (jnp.asarray(dim, jnp.int32) - 2)
    cur3 = cur.reshape(BH, 1, D)

    grid_spec = pltpu.PrefetchScalarGridSpec(
        num_scalar_prefetch=1,
        grid=(NSTEP,),
        in_specs=[pl.BlockSpec((BH, 1, D), lambda i, pos_ref: (0, 0, 0))],
        out_specs=pl.BlockSpec(memory_space=pl.ANY),
        scratch_shapes=[pltpu.VMEM((BPD, S, D), jnp.float32)] * NBUF
        + [pltpu.SemaphoreType.DMA] * NBUF,
    )
    out = pl.pallas_call(
        _tc_body,
        grid_spec=grid_spec,
        out_shape=jax.ShapeDtypeStruct((BH, S, D), jnp.float32),
    )(pos, cur3)
    return out.reshape(B, H, S, D)
